# baseline (speedup 1.0000x reference)
import functools

import jax
import jax.numpy as jnp
from jax import lax
from jax.experimental import pallas as pl
from jax.experimental.pallas import tpu as pltpu
from jax.experimental.pallas import tpu_sc as plsc

VOCAB = 1000000
EMB_DIM = 64
BATCH = 16384

NC = 2
NS = 16
NW = NC * NS

B_PER_W = 416
N_SC = NW * B_PER_W
N_TC = BATCH - N_SC
K = 16
RPT = 8

_mesh = plsc.VectorSubcoreMesh(core_axis_name="c", subcore_axis_name="s")


@functools.partial(
    pl.kernel,
    mesh=_mesh,
    out_type=jax.ShapeDtypeStruct((N_SC, EMB_DIM), jnp.float32),
    scratch_types=[
        pltpu.VMEM_SHARED((N_SC,), jnp.int32),
        pltpu.SMEM((B_PER_W,), jnp.int32),
        pltpu.VMEM((B_PER_W, EMB_DIM), jnp.float32),
        pltpu.SemaphoreType.DMA,
    ],
)
def _sc_gather(table_hbm, idx_hbm, out_hbm, idx_sp, idx_s, rows_v, sem):
    cid = lax.axis_index("c")
    sid = lax.axis_index("s")
    wid = sid * NC + cid
    base = wid * B_PER_W

    @pl.when(sid == 0)
    def _():
        pltpu.sync_copy(idx_hbm, idx_sp)

    plsc.subcore_barrier()
    pltpu.sync_copy(idx_sp.at[pl.ds(base, B_PER_W)], idx_s)

    def chunk(g, carry):
        row0 = g * K
        copies = [
            pltpu.async_copy(
                table_hbm.at[idx_s[row0 + j]], rows_v.at[row0 + j], sem
            )
            for j in range(K)
        ]
        for cp in copies:
            cp.wait()
        return carry

    lax.fori_loop(0, B_PER_W // K, chunk, 0)
    pltpu.sync_copy(rows_v, out_hbm.at[pl.ds(base, B_PER_W)])


def _tc_body(idx_ref, table_any, out_ref, buf, sem):
    i = pl.program_id(0)
    copies = [
        pltpu.make_async_copy(
            table_any.at[pl.ds(idx_ref[i * RPT + k], 1)],
            buf.at[pl.ds(k, 1)],
            sem,
        )
        for k in range(RPT)
    ]
    for cp in copies:
        cp.start()
    for cp in copies:
        cp.wait()
    out_ref[...] = buf[...]


_tc_gather = pl.pallas_call(
    _tc_body,
    grid_spec=pltpu.PrefetchScalarGridSpec(
        num_scalar_prefetch=1,
        grid=(N_TC // RPT,),
        in_specs=[pl.BlockSpec(memory_space=pl.ANY)],
        out_specs=pl.BlockSpec((RPT, EMB_DIM), lambda i, idx_ref: (i, 0)),
        scratch_shapes=[
            pltpu.VMEM((RPT, EMB_DIM), jnp.float32),
            pltpu.SemaphoreType.DMA,
        ],
    ),
    out_shape=jax.ShapeDtypeStruct((N_TC, EMB_DIM), jnp.float32),
)


def kernel(indices, table):
    idx = indices.astype(jnp.int32)
    out_sc = _sc_gather(table, idx[:N_SC])
    out_tc = _tc_gather(idx[N_SC:], table)
    return jnp.concatenate([out_sc, out_tc], axis=0)


# speedup vs baseline: 1.6742x; 1.6742x over previous
import functools

import jax
import jax.numpy as jnp
from jax import lax
from jax.experimental import pallas as pl
from jax.experimental.pallas import tpu as pltpu
from jax.experimental.pallas import tpu_sc as plsc

VOCAB = 1000000
EMB_DIM = 64
BATCH = 16384

NC = 2
NS = 16
NW = NC * NS
B_PER_W = BATCH // NW
K = 16

_mesh = plsc.VectorSubcoreMesh(core_axis_name="c", subcore_axis_name="s")


@functools.partial(
    pl.kernel,
    mesh=_mesh,
    out_type=jax.ShapeDtypeStruct((BATCH, EMB_DIM), jnp.float32),
    scratch_types=[
        pltpu.VMEM_SHARED((BATCH,), jnp.int32),
        pltpu.SMEM((B_PER_W,), jnp.int32),
        pltpu.VMEM((B_PER_W, EMB_DIM), jnp.float32),
        pltpu.SemaphoreType.DMA,
    ],
)
def _gather_rows(table_hbm, idx_hbm, out_hbm, idx_sp, idx_s, rows_v, sem):
    cid = lax.axis_index("c")
    sid = lax.axis_index("s")
    wid = sid * NC + cid
    base = wid * B_PER_W

    @pl.when(sid == 0)
    def _():
        pltpu.sync_copy(idx_hbm, idx_sp)

    plsc.subcore_barrier()
    pltpu.sync_copy(idx_sp.at[pl.ds(base, B_PER_W)], idx_s)

    def chunk(g, carry):
        row0 = g * K
        copies = [
            pltpu.async_copy(
                table_hbm.at[idx_s[row0 + j]],
                rows_v.at[row0 + j],
                sem,
            )
            for j in range(K)
        ]
        for cp in copies:
            cp.wait()
        return carry

    lax.fori_loop(0, B_PER_W // K, chunk, 0)
    pltpu.sync_copy(rows_v, out_hbm.at[pl.ds(base, B_PER_W)])


def kernel(indices, table):
    return _gather_rows(table, indices.astype(jnp.int32))


# speedup vs baseline: 1.7276x; 1.0319x over previous
import functools

import jax
import jax.numpy as jnp
from jax import lax
from jax.experimental import pallas as pl
from jax.experimental.pallas import tpu as pltpu
from jax.experimental.pallas import tpu_sc as plsc

VOCAB = 1000000
EMB_DIM = 64
BATCH = 16384

NC = 2
NS = 16
NW = NC * NS
B_PER_W = BATCH // NW
K = 16

_mesh = plsc.VectorSubcoreMesh(core_axis_name="c", subcore_axis_name="s")


@functools.partial(
    pl.kernel,
    mesh=_mesh,
    out_type=jax.ShapeDtypeStruct((BATCH, EMB_DIM), jnp.float32),
    scratch_types=[
        pltpu.VMEM_SHARED((BATCH,), jnp.int32),
        pltpu.SMEM((B_PER_W,), jnp.int32),
        pltpu.VMEM((B_PER_W, EMB_DIM), jnp.float32),
        pltpu.SemaphoreType.DMA,
    ],
)
def _gather_rows(table_hbm, idx_hbm, out_hbm, idx_sp, idx_s, rows_v, sem):
    cid = lax.axis_index("c")
    sid = lax.axis_index("s")
    wid = sid * NC + cid
    base = wid * B_PER_W

    @pl.when(sid == 0)
    def _():
        pltpu.sync_copy(idx_hbm, idx_sp)

    plsc.subcore_barrier()
    pltpu.sync_copy(idx_sp.at[pl.ds(base, B_PER_W)], idx_s)

    def chunk(g, carry):
        row0 = g * K
        for j in range(K):
            pltpu.async_copy(
                table_hbm.at[idx_s[row0 + j]],
                rows_v.at[row0 + j],
                sem,
            )

        @pl.when(g > 0)
        def _():
            prev0 = (g - 1) * K
            for j in range(K):
                pltpu.make_async_copy(
                    table_hbm.at[0], rows_v.at[prev0 + j], sem
                ).wait()
        return carry

    lax.fori_loop(0, B_PER_W // K, chunk, 0)
    for j in range(K):
        pltpu.make_async_copy(
            table_hbm.at[0], rows_v.at[B_PER_W - K + j], sem
        ).wait()
    pltpu.sync_copy(rows_v, out_hbm.at[pl.ds(base, B_PER_W)])


def kernel(indices, table):
    return _gather_rows(table, indices.astype(jnp.int32))


# speedup vs baseline: 1.7406x; 1.0076x over previous
import functools

import jax
import jax.numpy as jnp
from jax import lax
from jax.experimental import pallas as pl
from jax.experimental.pallas import tpu as pltpu
from jax.experimental.pallas import tpu_sc as plsc

VOCAB = 1000000
EMB_DIM = 64
BATCH = 16384

NC = 2
NS = 16
NW = NC * NS
B_PER_W = BATCH // NW
K = 16

_mesh = plsc.VectorSubcoreMesh(core_axis_name="c", subcore_axis_name="s")


@functools.partial(
    pl.kernel,
    mesh=_mesh,
    out_type=jax.ShapeDtypeStruct((BATCH, EMB_DIM), jnp.float32),
    scratch_types=[
        pltpu.VMEM_SHARED((BATCH,), jnp.int32),
        pltpu.SMEM((B_PER_W,), jnp.int32),
        pltpu.VMEM((B_PER_W, EMB_DIM), jnp.float32),
        pltpu.SemaphoreType.DMA,
    ],
)
def _gather_rows(table_hbm, idx_hbm, out_hbm, idx_sp, idx_s, rows_v, sem):
    cid = lax.axis_index("c")
    sid = lax.axis_index("s")
    wid = sid * NC + cid
    base = wid * B_PER_W

    @pl.when(sid == 0)
    def _():
        pltpu.sync_copy(idx_hbm, idx_sp)

    plsc.subcore_barrier()
    pltpu.sync_copy(idx_sp.at[pl.ds(base, B_PER_W)], idx_s)

    def chunk(g, carry):
        row0 = g * K
        for j in range(K):
            pltpu.async_copy(
                table_hbm.at[idx_s[row0 + j]],
                rows_v.at[row0 + j],
                sem,
            )

        @pl.when(g > 1)
        def _():
            prev0 = (g - 2) * K
            for j in range(K):
                pltpu.make_async_copy(
                    table_hbm.at[0], rows_v.at[prev0 + j], sem
                ).wait()
        return carry

    lax.fori_loop(0, B_PER_W // K, chunk, 0)
    for j in range(2 * K):
        pltpu.make_async_copy(
            table_hbm.at[0], rows_v.at[B_PER_W - 2 * K + j], sem
        ).wait()
    pltpu.sync_copy(rows_v, out_hbm.at[pl.ds(base, B_PER_W)])


def kernel(indices, table):
    return _gather_rows(table, indices.astype(jnp.int32))
